# Initial kernel scaffold; baseline (speedup 1.0000x reference)
import jax
import jax.numpy as jnp
from jax.experimental import pallas as pl


def kernel(input_ids, word_embeddings):
    raise NotImplementedError("write your pallas kernel here")


# speedup vs baseline: 2.9553x; 2.9553x over previous
import functools

import jax
import jax.numpy as jnp
from jax import lax
from jax.experimental import pallas as pl
from jax.experimental.pallas import tpu as pltpu
from jax.experimental.pallas import tpu_sc as plsc

NUM_WORKERS = 32
CHUNK = 128


def _build_sc_gather(n_per_w: int, d: int, n_total: int):
    n_chunks = n_per_w // CHUNK
    mesh = plsc.VectorSubcoreMesh(core_axis_name="c", subcore_axis_name="s")

    @functools.partial(
        pl.kernel,
        mesh=mesh,
        out_type=jax.ShapeDtypeStruct((n_total, d), jnp.float32),
        scratch_types=[
            pltpu.VMEM((n_chunks, CHUNK), jnp.int32),
            pltpu.VMEM((CHUNK, d), jnp.float32),
            pltpu.SemaphoreType.DMA,
        ],
    )
    def sc_gather(idx_hbm, table_hbm, out_hbm, idx_v, rows_v, sem):
        wid = lax.axis_index("s") * 2 + lax.axis_index("c")
        base = wid * n_per_w
        pltpu.sync_copy(idx_hbm.at[wid], idx_v)
        for c in range(n_chunks):
            pltpu.async_copy(table_hbm.at[idx_v.at[c]], rows_v, sem).wait()
            pltpu.sync_copy(rows_v, out_hbm.at[pl.ds(base + c * CHUNK, CHUNK)])

    return sc_gather


def kernel(input_ids, word_embeddings):
    b, h = input_ids.shape
    v, d = word_embeddings.shape
    n_total = b * h
    n_per_w = n_total // NUM_WORKERS
    idx = input_ids.reshape(NUM_WORKERS, n_per_w // CHUNK, CHUNK).astype(jnp.int32)
    out = _build_sc_gather(n_per_w, d, n_total)(idx, word_embeddings)
    return out.reshape(b, h, d)


# speedup vs baseline: 3.3332x; 1.1279x over previous
import functools

import jax
import jax.numpy as jnp
from jax import lax
from jax.experimental import pallas as pl
from jax.experimental.pallas import tpu as pltpu
from jax.experimental.pallas import tpu_sc as plsc

NUM_WORKERS = 32
CHUNK = 128


NBUF = 6


def _build_sc_gather(n_per_w: int, d: int, n_total: int):
    n_chunks = n_per_w // CHUNK
    mesh = plsc.VectorSubcoreMesh(core_axis_name="c", subcore_axis_name="s")

    @functools.partial(
        pl.kernel,
        mesh=mesh,
        out_type=jax.ShapeDtypeStruct((n_total, d), jnp.float32),
        scratch_types=[
            pltpu.VMEM((n_chunks, CHUNK), jnp.int32),
        ]
        + [pltpu.VMEM((CHUNK, d), jnp.float32) for _ in range(NBUF)]
        + [pltpu.SemaphoreType.DMA for _ in range(NBUF)],
    )
    def sc_gather(idx_hbm, table_hbm, out_hbm, idx_v, *bufs_and_sems):
        bufs = bufs_and_sems[:NBUF]
        sems = bufs_and_sems[NBUF:]
        wid = lax.axis_index("s") * 2 + lax.axis_index("c")
        base = wid * n_per_w
        pltpu.sync_copy(idx_hbm.at[wid], idx_v)
        for b in range(min(NBUF, n_chunks)):
            pltpu.async_copy(table_hbm.at[idx_v.at[b]], bufs[b], sems[b])
        for c in range(n_chunks):
            s = c % NBUF
            pltpu.make_async_copy(table_hbm.at[idx_v.at[c]], bufs[s], sems[s]).wait()
            pltpu.sync_copy(bufs[s], out_hbm.at[pl.ds(base + c * CHUNK, CHUNK)])
            nxt = c + NBUF
            if nxt < n_chunks:
                pltpu.async_copy(table_hbm.at[idx_v.at[nxt]], bufs[s], sems[s])

    return sc_gather


def kernel(input_ids, word_embeddings):
    b, h = input_ids.shape
    v, d = word_embeddings.shape
    n_total = b * h
    n_per_w = n_total // NUM_WORKERS
    idx = input_ids.reshape(NUM_WORKERS, n_per_w // CHUNK, CHUNK).astype(jnp.int32)
    out = _build_sc_gather(n_per_w, d, n_total)(idx, word_embeddings)
    return out.reshape(b, h, d)


# speedup vs baseline: 5.8879x; 1.7665x over previous
import functools

import jax
import jax.numpy as jnp
from jax import lax
from jax.experimental import pallas as pl
from jax.experimental.pallas import tpu as pltpu
from jax.experimental.pallas import tpu_sc as plsc

NUM_WORKERS = 32
NBUF = 8


def _build_sc_gather(b: int, h: int, d: int):
    b_per_w = b // NUM_WORKERS
    mesh = plsc.VectorSubcoreMesh(core_axis_name="c", subcore_axis_name="s")

    @functools.partial(
        pl.kernel,
        mesh=mesh,
        out_type=jax.ShapeDtypeStruct((b, h, d), jnp.float32),
        scratch_types=[
            pltpu.VMEM((b_per_w, h), jnp.int32),
        ]
        + [pltpu.VMEM((h, d), jnp.float32) for _ in range(NBUF)]
        + [pltpu.SemaphoreType.DMA for _ in range(NBUF)],
    )
    def sc_gather(idx_hbm, table_hbm, out_hbm, idx_v, *bufs_and_sems):
        bufs = bufs_and_sems[:NBUF]
        sems = bufs_and_sems[NBUF:]
        wid = lax.axis_index("s") * 2 + lax.axis_index("c")
        base = wid * b_per_w
        pltpu.sync_copy(idx_hbm.at[wid], idx_v)
        for r in range(min(NBUF, b_per_w)):
            pltpu.async_copy(table_hbm.at[idx_v.at[r]], bufs[r], sems[r])
        for r in range(b_per_w):
            s = r % NBUF
            pltpu.make_async_copy(table_hbm.at[idx_v.at[r]], bufs[s], sems[s]).wait()
            pltpu.sync_copy(bufs[s], out_hbm.at[base + r])
            nxt = r + NBUF
            if nxt < b_per_w:
                pltpu.async_copy(table_hbm.at[idx_v.at[nxt]], bufs[s], sems[s])

    return sc_gather


def kernel(input_ids, word_embeddings):
    b, h = input_ids.shape
    v, d = word_embeddings.shape
    idx = input_ids.reshape(NUM_WORKERS, b // NUM_WORKERS, h).astype(jnp.int32)
    return _build_sc_gather(b, h, d)(idx, word_embeddings)
